# baseline (speedup 1.0000x reference)
from functools import partial

import jax
import jax.numpy as jnp
from jax import lax
from jax.experimental import pallas as pl
from jax.experimental.pallas import tpu as pltpu
from jax.experimental.pallas import tpu_sc as plsc

ENERGY_MOMENTUM = 0.9
BASE_PROB = 0.1
ENERGY_SCALE = 2.0
LAYER_IDX = 0

_ROW_LEN = 32 * 128
_NW = 32
_SC_ROWS = 4096
_SC_R = 8
_TC_CHUNK = 512


def _tc_partials_kernel(x_ref, part_ref, acc_ref, *, n_steps):
    i = pl.program_id(0)

    @pl.when(i == 0)
    def _init():
        acc_ref[...] = jnp.zeros_like(acc_ref)

    x = x_ref[...]
    acc_ref[0, :] += jnp.sum(x, axis=(0, 2))
    acc_ref[1, :] += jnp.sum(x * x, axis=(0, 2))

    @pl.when(i == n_steps - 1)
    def _emit():
        part_ref[...] = acc_ref[...]


def _sc_body(x_hbm, out_hbm, buf0, buf1, acc, sem0, sem1):
    rows_per_w = _SC_ROWS // _NW
    n_chunks = rows_per_w // _SC_R
    chunk = _SC_R * _ROW_LEN
    tc_rows = 16384 - _SC_ROWS

    wid = lax.axis_index("s") * 2 + lax.axis_index("c")
    base = (tc_rows + wid * rows_per_w) * _ROW_LEN

    bufs = (buf0, buf1)
    sems = (sem0, sem1)

    for h in range(64):
        acc[h, :] = jnp.zeros((16,), jnp.float32)

    def start(c, b):
        pltpu.make_async_copy(
            x_hbm.at[pl.ds(base + c * chunk, chunk)], bufs[b], sems[b]
        ).start()

    def wait(b):
        pltpu.make_async_copy(
            x_hbm.at[pl.ds(base, chunk)], bufs[b], sems[b]
        ).wait()

    def compute(b):
        buf = bufs[b]
        for h in range(32):
            def row_body(r2, carry):
                s = list(carry[0])
                q = list(carry[1])
                base = 2 * r2 * _ROW_LEN + h * 128
                for rr in range(2):
                    off = base + rr * _ROW_LEN
                    for j in range(8):
                        v = buf[pl.ds(off + j * 16, 16)]
                        k = j % 4
                        s[k] = s[k] + v
                        q[k] = q[k] + v * v
                return tuple(s), tuple(q)

            z = jnp.zeros((16,), jnp.float32)
            (s0, s1, s2, s3), (q0, q1, q2, q3) = lax.fori_loop(
                0, _SC_R // 2, row_body, ((z, z, z, z), (z, z, z, z))
            )
            acc[h, :] = acc[h, :] + ((s0 + s1) + (s2 + s3))
            acc[32 + h, :] = acc[32 + h, :] + ((q0 + q1) + (q2 + q3))

    start(0, 0)
    start(1, 1)

    def outer(i, carry):
        c = i * 2
        for b in range(2):
            cb = c + b
            wait(b)
            compute(b)

            @pl.when(cb + 2 < n_chunks)
            def _next():
                start(cb + 2, b)
        return carry

    lax.fori_loop(0, n_chunks // 2, outer, 0)

    pltpu.sync_copy(acc, out_hbm.at[wid])


def _epilogue_kernel(tc_ref, sc_ref, he_ref, probs_ref, *, n_total):
    tc = tc_ref[...]
    sc = sc_ref[...]
    ssum = tc[0, :] + jnp.sum(sc[:, 0:32, :], axis=(0, 2))
    ssq = tc[1, :] + jnp.sum(sc[:, 32:64, :], axis=(0, 2))
    n = jnp.float32(n_total)
    head_var = (ssq - ssum * ssum / n) / (n - 1.0)
    mx = jnp.max(head_var)
    head_var = jnp.where(mx > 0, head_var / (mx + 1e-08), head_var)

    he = he_ref[...]
    new_row = ENERGY_MOMENTUM * he[LAYER_IDX, :] + (1.0 - ENERGY_MOMENTUM) * head_var
    row_ids = lax.broadcasted_iota(jnp.int32, he.shape, 0)
    new_energy = jnp.where(row_ids == LAYER_IDX, new_row[None, :], he)

    inv = 1.0 / (new_energy + 0.1)
    inv = inv / jnp.max(inv)
    probs = BASE_PROB * (1.0 + ENERGY_SCALE * inv)
    probs_ref[...] = jnp.clip(probs, 0.0, 1.0)


def kernel(output, head_energy):
    B, T, H, D = output.shape
    rows = B * T
    tc_rows = rows - _SC_ROWS
    n_tc_steps = tc_rows // _TC_CHUNK
    n_total = rows * D

    x3 = output.reshape(rows, H, D)
    xf = output.reshape(rows * H * D)

    sc_kernel = pl.kernel(
        _sc_body,
        out_type=jax.ShapeDtypeStruct((_NW, 64, 16), jnp.float32),
        mesh=plsc.VectorSubcoreMesh(
            core_axis_name="c", subcore_axis_name="s", num_cores=2, num_subcores=16
        ),
        scratch_types=[
            pltpu.VMEM((_SC_R * _ROW_LEN,), jnp.float32),
            pltpu.VMEM((_SC_R * _ROW_LEN,), jnp.float32),
            pltpu.VMEM((64, 16), jnp.float32),
            pltpu.SemaphoreType.DMA,
            pltpu.SemaphoreType.DMA,
        ],
    )
    sc_partials = sc_kernel(xf)

    tc_partials = pl.pallas_call(
        partial(_tc_partials_kernel, n_steps=n_tc_steps),
        grid=(n_tc_steps,),
        in_specs=[pl.BlockSpec((_TC_CHUNK, H, D), lambda i: (i, 0, 0))],
        out_specs=pl.BlockSpec((8, H), lambda i: (0, 0)),
        out_shape=jax.ShapeDtypeStruct((8, H), jnp.float32),
        scratch_shapes=[pltpu.VMEM((8, H), jnp.float32)],
    )(x3)

    return pl.pallas_call(
        partial(_epilogue_kernel, n_total=n_total),
        in_specs=[
            pl.BlockSpec((8, H), lambda: (0, 0)),
            pl.BlockSpec(sc_partials.shape, lambda: (0, 0, 0)),
            pl.BlockSpec(head_energy.shape, lambda: (0, 0)),
        ],
        out_specs=pl.BlockSpec(head_energy.shape, lambda: (0, 0)),
        out_shape=jax.ShapeDtypeStruct(head_energy.shape, jnp.float32),
    )(tc_partials, sc_partials, head_energy)


# speedup vs baseline: 1.0386x; 1.0386x over previous
from functools import partial

import jax
import jax.numpy as jnp
from jax.experimental import pallas as pl
from jax.experimental.pallas import tpu as pltpu

ENERGY_MOMENTUM = 0.9
BASE_PROB = 0.1
ENERGY_SCALE = 2.0
LAYER_IDX = 0

_CHUNK = 256


def _var_probs_kernel(x_ref, he_ref, probs_ref, acc_ref, *, n_steps, n_total):
    i = pl.program_id(0)

    @pl.when(i == 0)
    def _init():
        acc_ref[...] = jnp.zeros_like(acc_ref)

    x = x_ref[...]
    acc_ref[0, :] += jnp.sum(x, axis=(0, 2))
    acc_ref[1, :] += jnp.sum(x * x, axis=(0, 2))

    @pl.when(i == n_steps - 1)
    def _epilogue():
        ssum = acc_ref[0, :]
        ssq = acc_ref[1, :]
        n = jnp.float32(n_total)
        head_var = (ssq - ssum * ssum / n) / (n - 1.0)
        mx = jnp.max(head_var)
        head_var = jnp.where(mx > 0, head_var / (mx + 1e-08), head_var)

        he = he_ref[...]
        new_row = ENERGY_MOMENTUM * he[LAYER_IDX, :] + (1.0 - ENERGY_MOMENTUM) * head_var
        row_ids = jax.lax.broadcasted_iota(jnp.int32, he.shape, 0)
        new_energy = jnp.where(row_ids == LAYER_IDX, new_row[None, :], he)

        inv = 1.0 / (new_energy + 0.1)
        inv = inv / jnp.max(inv)
        probs = BASE_PROB * (1.0 + ENERGY_SCALE * inv)
        probs_ref[...] = jnp.clip(probs, 0.0, 1.0)


def kernel(output, head_energy):
    B, T, H, D = output.shape
    x = output.reshape(B * T, H, D)
    rows = B * T
    n_steps = rows // _CHUNK
    n_total = rows * D

    return pl.pallas_call(
        partial(_var_probs_kernel, n_steps=n_steps, n_total=n_total),
        grid=(n_steps,),
        in_specs=[
            pl.BlockSpec((_CHUNK, H, D), lambda i: (i, 0, 0)),
            pl.BlockSpec(head_energy.shape, lambda i: (0, 0)),
        ],
        out_specs=pl.BlockSpec(head_energy.shape, lambda i: (0, 0)),
        out_shape=jax.ShapeDtypeStruct(head_energy.shape, jnp.float32),
        scratch_shapes=[pltpu.VMEM((2, H), jnp.float32)],
    )(x, head_energy)


# speedup vs baseline: 1.2455x; 1.1992x over previous
from functools import partial

import jax
import jax.numpy as jnp
from jax.experimental import pallas as pl
from jax.experimental.pallas import tpu as pltpu

ENERGY_MOMENTUM = 0.9
BASE_PROB = 0.1
ENERGY_SCALE = 2.0
LAYER_IDX = 0

_CHUNK = 512


def _var_probs_kernel(x_ref, he_ref, probs_ref, acc_ref, *, n_steps, n_total):
    i = pl.program_id(0)

    @pl.when(i == 0)
    def _init():
        acc_ref[...] = jnp.zeros_like(acc_ref)

    x = x_ref[...]
    acc_ref[0, :, :] += jnp.sum(x, axis=0)
    acc_ref[1, :, :] += jnp.sum(x * x, axis=0)

    @pl.when(i == n_steps - 1)
    def _epilogue():
        ssum = jnp.sum(acc_ref[0, :, :], axis=1)
        ssq = jnp.sum(acc_ref[1, :, :], axis=1)
        n = jnp.float32(n_total)
        head_var = (ssq - ssum * ssum / n) / (n - 1.0)
        mx = jnp.max(head_var)
        head_var = jnp.where(mx > 0, head_var / (mx + 1e-08), head_var)

        he = he_ref[...]
        new_row = ENERGY_MOMENTUM * he[LAYER_IDX, :] + (1.0 - ENERGY_MOMENTUM) * head_var
        row_ids = jax.lax.broadcasted_iota(jnp.int32, he.shape, 0)
        new_energy = jnp.where(row_ids == LAYER_IDX, new_row[None, :], he)

        inv = 1.0 / (new_energy + 0.1)
        inv = inv / jnp.max(inv)
        probs = BASE_PROB * (1.0 + ENERGY_SCALE * inv)
        probs_ref[...] = jnp.clip(probs, 0.0, 1.0)


def kernel(output, head_energy):
    B, T, H, D = output.shape
    x = output.reshape(B * T, H, D)
    rows = B * T
    n_steps = rows // _CHUNK
    n_total = rows * D

    return pl.pallas_call(
        partial(_var_probs_kernel, n_steps=n_steps, n_total=n_total),
        grid=(n_steps,),
        in_specs=[
            pl.BlockSpec((_CHUNK, H, D), lambda i: (i, 0, 0)),
            pl.BlockSpec(head_energy.shape, lambda i: (0, 0)),
        ],
        out_specs=pl.BlockSpec(head_energy.shape, lambda i: (0, 0)),
        out_shape=jax.ShapeDtypeStruct(head_energy.shape, jnp.float32),
        scratch_shapes=[pltpu.VMEM((2, H, D), jnp.float32)],
    )(x, head_energy)
